# baseline (speedup 1.0000x reference)
import jax
import jax.numpy as jnp
from jax.experimental import pallas as pl

N_Q = 8
K = 1024
D = 128
BT = 600


def _rvq_block(x_ref, cb_ref, cbT_ref, out_ref):
    residual = x_ref[...]
    iota_k = jax.lax.broadcasted_iota(jnp.int32, (BT, K), 1)
    inds = []
    for q in range(N_Q):
        eT = cbT_ref[q]
        xe = jnp.dot(residual.astype(jnp.bfloat16), eT.astype(jnp.bfloat16),
                     preferred_element_type=jnp.float32)
        x_sq = jnp.sum(residual * residual, axis=1, keepdims=True)
        e_sq = jnp.sum(eT * eT, axis=0, keepdims=True)
        dist = -(x_sq - 2.0 * xe + e_sq)
        m = jnp.max(dist, axis=1, keepdims=True)
        ind = jnp.min(jnp.where(dist == m, iota_k, K), axis=1, keepdims=True)
        inds.append(ind)
        if q < N_Q - 1:
            onehot = (iota_k == ind).astype(jnp.float32)
            quant = jnp.dot(onehot, cb_ref[q], preferred_element_type=jnp.float32,
                            precision=jax.lax.Precision.HIGHEST)
            residual = residual - quant
    out_ref[...] = jnp.concatenate(inds, axis=1)


def kernel(x, codebooks):
    B, T, _ = x.shape
    n_tok = B * T
    xf = x.reshape(n_tok, D)
    cbT = codebooks.transpose(0, 2, 1)
    grid = (n_tok // BT,)
    codes = pl.pallas_call(
        _rvq_block,
        grid=grid,
        in_specs=[
            pl.BlockSpec((BT, D), lambda i: (i, 0)),
            pl.BlockSpec((N_Q, K, D), lambda i: (0, 0, 0)),
            pl.BlockSpec((N_Q, D, K), lambda i: (0, 0, 0)),
        ],
        out_specs=pl.BlockSpec((BT, N_Q), lambda i: (i, 0)),
        out_shape=jax.ShapeDtypeStruct((n_tok, N_Q), jnp.int32),
    )(xf, codebooks, cbT)
    return codes.reshape(B, T, N_Q).transpose(2, 0, 1).astype(jnp.int64)


# speedup vs baseline: 1.9068x; 1.9068x over previous
import jax
import jax.numpy as jnp
from jax.experimental import pallas as pl
from jax.experimental.pallas import tpu as pltpu

N_Q = 8
K = 1024
D = 128
BT = 1024
PAD_T = 24576


def _rvq_block(xT_ref, cb_ref, cbb_ref, cbT_ref, out_ref):
    rT = xT_ref[...]
    iota_k = jax.lax.broadcasted_iota(jnp.int32, (K, BT), 0)
    inds = []
    for q in range(N_Q):
        xeT = jnp.dot(cbb_ref[q], rT.astype(jnp.bfloat16),
                      preferred_element_type=jnp.float32)
        e = cb_ref[q]
        e_sq_half = 0.5 * jnp.sum(e * e, axis=1, keepdims=True)
        s = e_sq_half - xeT
        m = jnp.min(s, axis=0, keepdims=True)
        ind = jnp.min(jnp.where(s == m, iota_k, K), axis=0, keepdims=True)
        inds.append(ind)
        if q < N_Q - 1:
            idx_mod = jnp.broadcast_to(ind & 127, (D, BT))
            idx_div = jnp.broadcast_to(ind >> 7, (D, BT))
            quantT = jnp.zeros((D, BT), jnp.float32)
            for g in range(K // 128):
                part = jnp.take_along_axis(
                    cbT_ref[q, :, g * 128:(g + 1) * 128], idx_mod, axis=1)
                quantT = jnp.where(idx_div == g, part, quantT)
            rT = rT - quantT
    out_ref[...] = jnp.concatenate(inds, axis=0)


def kernel(x, codebooks):
    B, T, _ = x.shape
    n_tok = B * T
    xf = x.reshape(n_tok, D)
    xT = jnp.pad(xf, ((0, PAD_T - n_tok), (0, 0))).T
    cbT = codebooks.transpose(0, 2, 1)
    cbb = codebooks.astype(jnp.bfloat16)
    grid = (PAD_T // BT,)
    codes = pl.pallas_call(
        _rvq_block,
        grid=grid,
        in_specs=[
            pl.BlockSpec((D, BT), lambda i: (0, i)),
            pl.BlockSpec((N_Q, K, D), lambda i: (0, 0, 0)),
            pl.BlockSpec((N_Q, K, D), lambda i: (0, 0, 0)),
            pl.BlockSpec((N_Q, D, K), lambda i: (0, 0, 0)),
        ],
        out_specs=pl.BlockSpec((N_Q, BT), lambda i: (0, i)),
        out_shape=jax.ShapeDtypeStruct((N_Q, PAD_T), jnp.int32),
        compiler_params=pltpu.CompilerParams(
            dimension_semantics=("arbitrary",)),
    )(xT, codebooks, cbb, cbT)
    return codes[:, :n_tok].reshape(N_Q, B, T).astype(jnp.int64)


# speedup vs baseline: 2.2036x; 1.1557x over previous
import jax
import jax.numpy as jnp
from jax.experimental import pallas as pl
from jax.experimental.pallas import tpu as pltpu

N_Q = 8
K = 1024
D = 128
BT = 1024
H = 512
PAD_T = 24576


def _stage_argmin(q, r, cbb_ref, idx_ref, e_sq_half):
    xeT = jnp.dot(cbb_ref[q], r.astype(jnp.bfloat16),
                  preferred_element_type=jnp.float32)
    s = e_sq_half - xeT
    m = jnp.min(s, axis=0, keepdims=True)
    oh = (s == m).astype(jnp.bfloat16)
    p = jnp.dot(idx_ref[...], oh, preferred_element_type=jnp.float32)
    ind = (p[0:1, :] * 256.0 + p[1:2, :]).astype(jnp.int32)
    return ind


def _lookup(q, ind, cbT_ref):
    idx_mod = jnp.broadcast_to(ind & 127, (D, H))
    idx_div = jnp.broadcast_to(ind >> 7, (D, H))
    quantT = jnp.zeros((D, H), jnp.float32)
    for g in range(K // 128):
        part = jnp.take_along_axis(
            cbT_ref[q, :, g * 128:(g + 1) * 128], idx_mod, axis=1)
        quantT = jnp.where(idx_div == g, part, quantT)
    return quantT


def _rvq_block(xT_ref, cb_ref, cbb_ref, cbT_ref, idx_ref, out_ref):
    chains = [xT_ref[:, c * H:(c + 1) * H] for c in range(BT // H)]
    inds = [[] for _ in chains]
    for q in range(N_Q):
        e = cb_ref[q]
        e_sq_half = 0.5 * jnp.sum(e * e, axis=1, keepdims=True)
        for c, r in enumerate(chains):
            ind = _stage_argmin(q, r, cbb_ref, idx_ref, e_sq_half)
            inds[c].append(ind)
            if q < N_Q - 1:
                chains[c] = r - _lookup(q, ind, cbT_ref)
    out_ref[...] = jnp.concatenate(
        [jnp.concatenate(ii, axis=0) for ii in inds], axis=1)


def kernel(x, codebooks):
    B, T, _ = x.shape
    n_tok = B * T
    xf = x.reshape(n_tok, D)
    xT = jnp.pad(xf, ((0, PAD_T - n_tok), (0, 0))).T
    cbT = codebooks.transpose(0, 2, 1)
    cbb = codebooks.astype(jnp.bfloat16)
    ks = jnp.arange(K, dtype=jnp.int32)
    idx_tab = jnp.zeros((8, K), jnp.bfloat16)
    idx_tab = idx_tab.at[0].set((ks >> 8).astype(jnp.bfloat16))
    idx_tab = idx_tab.at[1].set((ks & 255).astype(jnp.bfloat16))
    grid = (PAD_T // BT,)
    codes = pl.pallas_call(
        _rvq_block,
        grid=grid,
        in_specs=[
            pl.BlockSpec((D, BT), lambda i: (0, i)),
            pl.BlockSpec((N_Q, K, D), lambda i: (0, 0, 0)),
            pl.BlockSpec((N_Q, K, D), lambda i: (0, 0, 0)),
            pl.BlockSpec((N_Q, D, K), lambda i: (0, 0, 0)),
            pl.BlockSpec((8, K), lambda i: (0, 0)),
        ],
        out_specs=pl.BlockSpec((N_Q, BT), lambda i: (0, i)),
        out_shape=jax.ShapeDtypeStruct((N_Q, PAD_T), jnp.int32),
        compiler_params=pltpu.CompilerParams(
            dimension_semantics=("arbitrary",)),
    )(xT, codebooks, cbb, cbT, idx_tab)
    return codes[:, :n_tok].reshape(N_Q, B, T).astype(jnp.int64)


# speedup vs baseline: 2.2444x; 1.0185x over previous
import jax
import jax.numpy as jnp
from jax.experimental import pallas as pl
from jax.experimental.pallas import tpu as pltpu

N_Q = 8
K = 1024
D = 128
BT = 2048
H = 512
PAD_T = 24576


def _stage_argmin(q, r, cbb_ref, idx_ref, e_sq_half):
    xeT = jnp.dot(cbb_ref[q], r.astype(jnp.bfloat16),
                  preferred_element_type=jnp.float32)
    s = e_sq_half - xeT
    m = jnp.min(s, axis=0, keepdims=True)
    oh = (s == m).astype(jnp.bfloat16)
    p = jnp.dot(idx_ref[...], oh, preferred_element_type=jnp.float32)
    ind = (p[0:1, :] * 256.0 + p[1:2, :]).astype(jnp.int32)
    return ind


def _lookup(q, ind, cbT_ref):
    idx_mod = jnp.broadcast_to(ind & 127, (D, H))
    idx_div = jnp.broadcast_to(ind >> 7, (D, H))
    quantT = jnp.zeros((D, H), jnp.float32)
    for g in range(K // 128):
        part = jnp.take_along_axis(
            cbT_ref[q, :, g * 128:(g + 1) * 128], idx_mod, axis=1)
        quantT = jnp.where(idx_div == g, part, quantT)
    return quantT


def _rvq_block(xT_ref, cb_ref, cbb_ref, cbT_ref, idx_ref, out_ref):
    chains = [xT_ref[:, c * H:(c + 1) * H] for c in range(BT // H)]
    inds = [[] for _ in chains]
    for q in range(N_Q):
        e = cb_ref[q]
        e_sq_half = 0.5 * jnp.sum(e * e, axis=1, keepdims=True)
        for c, r in enumerate(chains):
            ind = _stage_argmin(q, r, cbb_ref, idx_ref, e_sq_half)
            inds[c].append(ind)
            if q < N_Q - 1:
                chains[c] = r - _lookup(q, ind, cbT_ref)
    out_ref[...] = jnp.concatenate(
        [jnp.concatenate(ii, axis=0) for ii in inds], axis=1)


def kernel(x, codebooks):
    B, T, _ = x.shape
    n_tok = B * T
    xf = x.reshape(n_tok, D)
    xT = jnp.pad(xf, ((0, PAD_T - n_tok), (0, 0))).T
    cbT = codebooks.transpose(0, 2, 1)
    cbb = codebooks.astype(jnp.bfloat16)
    ks = jnp.arange(K, dtype=jnp.int32)
    idx_tab = jnp.zeros((8, K), jnp.bfloat16)
    idx_tab = idx_tab.at[0].set((ks >> 8).astype(jnp.bfloat16))
    idx_tab = idx_tab.at[1].set((ks & 255).astype(jnp.bfloat16))
    grid = (PAD_T // BT,)
    codes = pl.pallas_call(
        _rvq_block,
        grid=grid,
        in_specs=[
            pl.BlockSpec((D, BT), lambda i: (0, i)),
            pl.BlockSpec((N_Q, K, D), lambda i: (0, 0, 0)),
            pl.BlockSpec((N_Q, K, D), lambda i: (0, 0, 0)),
            pl.BlockSpec((N_Q, D, K), lambda i: (0, 0, 0)),
            pl.BlockSpec((8, K), lambda i: (0, 0)),
        ],
        out_specs=pl.BlockSpec((N_Q, BT), lambda i: (0, i)),
        out_shape=jax.ShapeDtypeStruct((N_Q, PAD_T), jnp.int32),
        compiler_params=pltpu.CompilerParams(
            dimension_semantics=("arbitrary",)),
    )(xT, codebooks, cbb, cbT, idx_tab)
    return codes[:, :n_tok].reshape(N_Q, B, T).astype(jnp.int64)


# speedup vs baseline: 3.2773x; 1.4602x over previous
import jax
import jax.numpy as jnp
from jax.experimental import pallas as pl
from jax.experimental.pallas import tpu as pltpu

N_Q = 8
K = 1024
D = 128
BT = 2048
H = 512
PAD_T = 24576


def _rvq_block(xT_ref, cb_ref, cbb_ref, cb3_ref, idx_ref, out_ref):
    chains = [xT_ref[:, c * H:(c + 1) * H] for c in range(BT // H)]
    inds = [[] for _ in chains]
    for q in range(N_Q):
        e = cb_ref[q]
        e_sq_half = 0.5 * jnp.sum(e * e, axis=1, keepdims=True)
        for c, r in enumerate(chains):
            xeT = jnp.dot(cbb_ref[q], r.astype(jnp.bfloat16),
                          preferred_element_type=jnp.float32)
            s = e_sq_half - xeT
            m = jnp.min(s, axis=0, keepdims=True)
            oh = (s == m).astype(jnp.bfloat16)
            p = jnp.dot(idx_ref[...], oh,
                        preferred_element_type=jnp.float32)
            ind = (p[0:1, :] * 256.0 + p[1:2, :]).astype(jnp.int32)
            inds[c].append(ind)
            if q < N_Q - 1:
                limbs = jnp.dot(cb3_ref[q], oh,
                                preferred_element_type=jnp.float32)
                quantT = (limbs[:D] + limbs[D:2 * D]) + limbs[2 * D:]
                chains[c] = r - quantT
    out_ref[...] = jnp.concatenate(
        [jnp.concatenate(ii, axis=0) for ii in inds], axis=1)


def kernel(x, codebooks):
    B, T, _ = x.shape
    n_tok = B * T
    xf = x.reshape(n_tok, D)
    xT = jnp.pad(xf, ((0, PAD_T - n_tok), (0, 0))).T
    cbb = codebooks.astype(jnp.bfloat16)
    hi = codebooks.astype(jnp.bfloat16)
    mid = (codebooks - hi.astype(jnp.float32)).astype(jnp.bfloat16)
    lo = (codebooks - hi.astype(jnp.float32)
          - mid.astype(jnp.float32)).astype(jnp.bfloat16)
    cb3 = jnp.concatenate([hi, mid, lo], axis=2)
    cb3 = cb3.transpose(0, 2, 1)
    ks = jnp.arange(K, dtype=jnp.int32)
    idx_tab = jnp.zeros((8, K), jnp.bfloat16)
    idx_tab = idx_tab.at[0].set((ks >> 8).astype(jnp.bfloat16))
    idx_tab = idx_tab.at[1].set((ks & 255).astype(jnp.bfloat16))
    grid = (PAD_T // BT,)
    codes = pl.pallas_call(
        _rvq_block,
        grid=grid,
        in_specs=[
            pl.BlockSpec((D, BT), lambda i: (0, i)),
            pl.BlockSpec((N_Q, K, D), lambda i: (0, 0, 0)),
            pl.BlockSpec((N_Q, K, D), lambda i: (0, 0, 0)),
            pl.BlockSpec((N_Q, 3 * D, K), lambda i: (0, 0, 0)),
            pl.BlockSpec((8, K), lambda i: (0, 0)),
        ],
        out_specs=pl.BlockSpec((N_Q, BT), lambda i: (0, i)),
        out_shape=jax.ShapeDtypeStruct((N_Q, PAD_T), jnp.int32),
        compiler_params=pltpu.CompilerParams(
            dimension_semantics=("arbitrary",)),
    )(xT, codebooks, cbb, cb3, idx_tab)
    return codes[:, :n_tok].reshape(N_Q, B, T).astype(jnp.int64)


# speedup vs baseline: 3.3276x; 1.0153x over previous
import jax
import jax.numpy as jnp
from jax.experimental import pallas as pl
from jax.experimental.pallas import tpu as pltpu

N_Q = 8
K = 1024
D = 128
BT = 2048
H = 512
PAD_T = 24576


def _rvq_block(xT_ref, cb_ref, cbb_ref, cb3_ref, idx_ref, out_ref):
    chains = [xT_ref[:, c * H:(c + 1) * H] for c in range(BT // H)]
    inds = [[] for _ in chains]
    for q in range(N_Q):
        e = cb_ref[q]
        e_sq_half = 0.5 * jnp.sum(e * e, axis=1, keepdims=True)
        for c, r in enumerate(chains):
            xeT = jnp.dot(cbb_ref[q], r.astype(jnp.bfloat16),
                          preferred_element_type=jnp.float32)
            s = e_sq_half - xeT
            m = jnp.min(s, axis=0, keepdims=True)
            oh = (s == m).astype(jnp.bfloat16)
            p = jnp.dot(idx_ref[...], oh,
                        preferred_element_type=jnp.float32)
            ind = (p[0:1, :] * 256.0 + p[1:2, :]).astype(jnp.int32)
            inds[c].append(ind)
            if q < N_Q - 1:
                limbs = jnp.dot(cb3_ref[q], oh,
                                preferred_element_type=jnp.float32)
                quantT = (limbs[:D] + limbs[D:2 * D]) + limbs[2 * D:]
                chains[c] = r - quantT
    out_ref[...] = jnp.concatenate(
        [jnp.concatenate(ii, axis=0) for ii in inds], axis=1)


def kernel(x, codebooks):
    B, T, _ = x.shape
    n_tok = B * T
    xf = x.reshape(n_tok, D)
    xT = jnp.pad(xf, ((0, PAD_T - n_tok), (0, 0))).T
    cbb = codebooks.astype(jnp.bfloat16)
    mask = jnp.uint32(0xFFFF0000)
    bits = jax.lax.bitcast_convert_type(codebooks, jnp.uint32)
    hi_f = jax.lax.bitcast_convert_type(bits & mask, jnp.float32)
    mid_full = codebooks - hi_f
    mbits = jax.lax.bitcast_convert_type(mid_full, jnp.uint32)
    mid_f = jax.lax.bitcast_convert_type(mbits & mask, jnp.float32)
    lo_f = mid_full - mid_f
    hi = hi_f.astype(jnp.bfloat16)
    mid = mid_f.astype(jnp.bfloat16)
    lo = lo_f.astype(jnp.bfloat16)
    cb3 = jnp.concatenate([hi, mid, lo], axis=2)
    cb3 = cb3.transpose(0, 2, 1)
    ks = jnp.arange(K, dtype=jnp.int32)
    idx_tab = jnp.zeros((8, K), jnp.bfloat16)
    idx_tab = idx_tab.at[0].set((ks >> 8).astype(jnp.bfloat16))
    idx_tab = idx_tab.at[1].set((ks & 255).astype(jnp.bfloat16))
    grid = (PAD_T // BT,)
    codes = pl.pallas_call(
        _rvq_block,
        grid=grid,
        in_specs=[
            pl.BlockSpec((D, BT), lambda i: (0, i)),
            pl.BlockSpec((N_Q, K, D), lambda i: (0, 0, 0)),
            pl.BlockSpec((N_Q, K, D), lambda i: (0, 0, 0)),
            pl.BlockSpec((N_Q, 3 * D, K), lambda i: (0, 0, 0)),
            pl.BlockSpec((8, K), lambda i: (0, 0)),
        ],
        out_specs=pl.BlockSpec((N_Q, BT), lambda i: (0, i)),
        out_shape=jax.ShapeDtypeStruct((N_Q, PAD_T), jnp.int32),
        compiler_params=pltpu.CompilerParams(
            dimension_semantics=("arbitrary",)),
    )(xT, codebooks, cbb, cb3, idx_tab)
    return codes[:, :n_tok].reshape(N_Q, B, T).astype(jnp.int64)
